# baseline (speedup 1.0000x reference)
import functools

import jax
import jax.numpy as jnp
from jax import lax
from jax.experimental import pallas as pl
from jax.experimental.pallas import tpu as pltpu
from jax.experimental.pallas import tpu_sc as plsc

D = 128
L = 16
CH = 64
NW = 32
E = 320000
CHUNKS = 158
EPW = CH * CHUNKS
EP = EPW * NW
TCH = NW * CHUNKS
N = 10000
D2 = D // 2
MHI = -65536
NP = 10240


def _lane_perm(v, idx):
    dnums = lax.GatherDimensionNumbers(
        offset_dims=(), collapsed_slice_dims=(0,), start_index_map=(0,))
    return lax.gather(v, idx[:, None], dnums, (1,),
                      mode=lax.GatherScatterMode.PROMISE_IN_BOUNDS)


def _make_ip_kernel():
    mesh = plsc.VectorSubcoreMesh(core_axis_name="c", subcore_axis_name="s")

    @functools.partial(
        pl.kernel,
        mesh=mesh,
        compiler_params=pltpu.CompilerParams(needs_layout_passes=False,
                                             use_tc_tiling_on_sc=False),
        out_type=jax.ShapeDtypeStruct((EP,), jnp.float32),
        scratch_types=[
            pltpu.VMEM((2, 2, CH), jnp.int32),
            pltpu.VMEM((2, CH, D2), jnp.int32),
            pltpu.VMEM((2, CH, D2), jnp.int32),
            pltpu.VMEM((2, CH), jnp.float32),
            pltpu.VMEM((CH // 2, L), jnp.float32),
            pltpu.VMEM_SHARED((NP, D2), jnp.int32),
            pltpu.SemaphoreType.DMA,
            pltpu.SemaphoreType.DMA,
            pltpu.SemaphoreType.DMA,
            pltpu.SemaphoreType.DMA,
            pltpu.SemaphoreType.DMA,
            pltpu.SemaphoreType.DMA,
        ],
    )
    def ip(x_hbm, idx_hbm, out_hbm,
           idx_v, rows_s, rows_d, out_v, pbuf, x_sh,
           sem_i0, sem_i1, sem_g0, sem_g1, sem_o0, sem_o1):
        wid = lax.axis_index("s") * 2 + lax.axis_index("c")
        t0 = wid * CHUNKS
        lane = lax.iota(jnp.int32, L)
        perm_idx = [lane ^ s for s in (1, 2, 4, 8)]
        sel_mask = [(lane & s) == 0 for s in (1, 2, 4, 8)]

        def merge(u, v, lvl):
            pu = _lane_perm(u, perm_idx[lvl])
            pv = _lane_perm(v, perm_idx[lvl])
            return (jnp.where(sel_mask[lvl], u, pv)
                    + jnp.where(sel_mask[lvl], pu, v))
        sem_i = (sem_i0, sem_i1)
        sem_g = (sem_g0, sem_g1)
        sem_o = (sem_o0, sem_o1)

        def issue_idx(c, b):
            pltpu.async_copy(idx_hbm.at[t0 + c], idx_v.at[b], sem_i[b])

        def wait_idx(b):
            pltpu.make_async_copy(idx_hbm.at[t0], idx_v.at[b],
                                  sem_i[b]).wait()

        NSPLIT = 4
        SW = CH // NSPLIT

        def issue_gather(b):
            for p in range(NSPLIT):
                sl = pl.ds(p * SW, SW)
                pltpu.async_copy(x_sh.at[idx_v.at[b, 0, sl]],
                                 rows_s.at[b, sl], sem_g[b])
                pltpu.async_copy(x_sh.at[idx_v.at[b, 1, sl]],
                                 rows_d.at[b, sl], sem_g[b])

        def wait_gather(b):
            for p in range(NSPLIT):
                sl = pl.ds(p * SW, SW)
                pltpu.make_async_copy(x_sh.at[idx_v.at[b, 0, sl]],
                                      rows_s.at[b, sl], sem_g[b]).wait()
                pltpu.make_async_copy(x_sh.at[idx_v.at[b, 1, sl]],
                                      rows_d.at[b, sl], sem_g[b]).wait()

        def issue_out(c, b):
            pltpu.async_copy(out_v.at[b],
                             out_hbm.at[pl.ds((t0 + c) * CH, CH)], sem_o[b])

        def wait_out(b):
            pltpu.make_async_copy(out_v.at[b], out_hbm.at[pl.ds(0, CH)],
                                  sem_o[b]).wait()

        def compute(b):
            rs, rd, ov = rows_s.at[b], rows_d.at[b], out_v.at[b]

            def edge_acc(e):
                a = None
                b = None
                for k in range(D2 // L):
                    sw = rs[e, pl.ds(k * L, L)]
                    dw = rd[e, pl.ds(k * L, L)]
                    sa = plsc.bitcast(sw & MHI, jnp.float32)
                    da = plsc.bitcast(dw & MHI, jnp.float32)
                    sb = plsc.bitcast(lax.shift_left(sw, 16), jnp.float32)
                    db = plsc.bitcast(lax.shift_left(dw, 16), jnp.float32)
                    a = sa * da if a is None else a + sa * da
                    b = sb * db if b is None else b + sb * db
                return a + b

            def pair_body(p, _):
                e0 = 2 * p
                pbuf[p] = merge(edge_acc(e0), edge_acc(e0 + 1), 0)
                return 0

            def fin_body(g, _):
                q0 = g * 8
                a = merge(pbuf[q0 + 0], pbuf[q0 + 1], 1)
                b = merge(pbuf[q0 + 2], pbuf[q0 + 3], 1)
                c = merge(pbuf[q0 + 4], pbuf[q0 + 5], 1)
                d = merge(pbuf[q0 + 6], pbuf[q0 + 7], 1)
                ov[pl.ds(g * L, L)] = merge(merge(a, b, 2), merge(c, d, 2), 3)
                return 0

            lax.fori_loop(0, CH // 2, pair_body, 0)
            lax.fori_loop(0, CH // L, fin_body, 0)

        sid = lax.axis_index("s")
        rpt = NP // 16
        pltpu.sync_copy(x_hbm.at[pl.ds(sid * rpt, rpt)],
                        x_sh.at[pl.ds(sid * rpt, rpt)])
        plsc.subcore_barrier()

        issue_idx(0, 0)
        issue_idx(1, 1)
        wait_idx(0)
        issue_gather(0)

        def pair_body(i, carry):
            for b in (0, 1):
                c = 2 * i + b
                nb = 1 - b
                wait_gather(b)

                @pl.when(c + 2 < CHUNKS)
                def _():
                    issue_idx(c + 2, b)

                @pl.when(c + 1 < CHUNKS)
                def _():
                    wait_idx(nb)
                    issue_gather(nb)

                @pl.when(c >= 2)
                def _():
                    wait_out(b)

                compute(b)
                issue_out(c, b)
            return carry

        lax.fori_loop(0, CHUNKS // 2, pair_body, 0)
        wait_out(0)
        wait_out(1)

    return ip


_ip_kernel = _make_ip_kernel()


@jax.jit
def kernel(x, edge_index):
    ei = edge_index.astype(jnp.int32)
    src = jnp.pad(ei[0], (0, EP - E)).reshape(TCH, CH)
    dst = jnp.pad(ei[1], (0, EP - E)).reshape(TCH, CH)
    idx_packed = jnp.stack([src, dst], axis=1)
    x_bf = jnp.pad(x, ((0, NP - N), (0, 0))).astype(jnp.bfloat16)
    x_i32 = lax.bitcast_convert_type(x_bf.reshape(NP, D2, 2), jnp.int32)
    out = _ip_kernel(x_i32, idx_packed)
    return out[:E]


# speedup vs baseline: 1.0726x; 1.0726x over previous
import functools

import jax
import jax.numpy as jnp
from jax import lax
from jax.experimental import pallas as pl
from jax.experimental.pallas import tpu as pltpu
from jax.experimental.pallas import tpu_sc as plsc

D = 128
L = 16
CH = 128
NW = 32
E = 320000
CHUNKS = 80
EPW = CH * CHUNKS
EP = EPW * NW
TCH = NW * CHUNKS
N = 10000
D2 = D // 2
MHI = -65536
NP = 10240


def _lane_perm(v, idx):
    dnums = lax.GatherDimensionNumbers(
        offset_dims=(), collapsed_slice_dims=(0,), start_index_map=(0,))
    return lax.gather(v, idx[:, None], dnums, (1,),
                      mode=lax.GatherScatterMode.PROMISE_IN_BOUNDS)


def _make_ip_kernel():
    mesh = plsc.VectorSubcoreMesh(core_axis_name="c", subcore_axis_name="s")

    @functools.partial(
        pl.kernel,
        mesh=mesh,
        compiler_params=pltpu.CompilerParams(needs_layout_passes=False,
                                             use_tc_tiling_on_sc=False),
        out_type=jax.ShapeDtypeStruct((EP,), jnp.float32),
        scratch_types=[
            pltpu.VMEM((2, 2, CH), jnp.int32),
            pltpu.VMEM((2, CH, D2), jnp.int32),
            pltpu.VMEM((2, CH, D2), jnp.int32),
            pltpu.VMEM((2, CH), jnp.float32),
            pltpu.VMEM((CH // 2, L), jnp.float32),
            pltpu.VMEM_SHARED((NP, D2), jnp.int32),
            pltpu.SemaphoreType.DMA,
            pltpu.SemaphoreType.DMA,
            pltpu.SemaphoreType.DMA,
            pltpu.SemaphoreType.DMA,
            pltpu.SemaphoreType.DMA,
            pltpu.SemaphoreType.DMA,
        ],
    )
    def ip(x_hbm, idx_hbm, out_hbm,
           idx_v, rows_s, rows_d, out_v, pbuf, x_sh,
           sem_i0, sem_i1, sem_g0, sem_g1, sem_o0, sem_o1):
        wid = lax.axis_index("s") * 2 + lax.axis_index("c")
        t0 = wid * CHUNKS
        lane = lax.iota(jnp.int32, L)
        perm_idx = [lane ^ s for s in (1, 2, 4, 8)]
        sel_mask = [(lane & s) == 0 for s in (1, 2, 4, 8)]

        def merge(u, v, lvl):
            pu = _lane_perm(u, perm_idx[lvl])
            pv = _lane_perm(v, perm_idx[lvl])
            return (jnp.where(sel_mask[lvl], u, pv)
                    + jnp.where(sel_mask[lvl], pu, v))
        sem_i = (sem_i0, sem_i1)
        sem_g = (sem_g0, sem_g1)
        sem_o = (sem_o0, sem_o1)

        def issue_idx(c, b):
            pltpu.async_copy(idx_hbm.at[t0 + c], idx_v.at[b], sem_i[b])

        def wait_idx(b):
            pltpu.make_async_copy(idx_hbm.at[t0], idx_v.at[b],
                                  sem_i[b]).wait()

        NSPLIT = 1
        SW = CH // NSPLIT

        def issue_gather(b):
            for p in range(NSPLIT):
                sl = pl.ds(p * SW, SW)
                pltpu.async_copy(x_sh.at[idx_v.at[b, 0, sl]],
                                 rows_s.at[b, sl], sem_g[b])
                pltpu.async_copy(x_sh.at[idx_v.at[b, 1, sl]],
                                 rows_d.at[b, sl], sem_g[b])

        def wait_gather(b):
            for p in range(NSPLIT):
                sl = pl.ds(p * SW, SW)
                pltpu.make_async_copy(x_sh.at[idx_v.at[b, 0, sl]],
                                      rows_s.at[b, sl], sem_g[b]).wait()
                pltpu.make_async_copy(x_sh.at[idx_v.at[b, 1, sl]],
                                      rows_d.at[b, sl], sem_g[b]).wait()

        def issue_out(c, b):
            pltpu.async_copy(out_v.at[b],
                             out_hbm.at[pl.ds((t0 + c) * CH, CH)], sem_o[b])

        def wait_out(b):
            pltpu.make_async_copy(out_v.at[b], out_hbm.at[pl.ds(0, CH)],
                                  sem_o[b]).wait()

        def compute(b):
            rs, rd, ov = rows_s.at[b], rows_d.at[b], out_v.at[b]

            def edge_acc(e):
                a = None
                b = None
                for k in range(D2 // L):
                    sw = rs[e, pl.ds(k * L, L)]
                    dw = rd[e, pl.ds(k * L, L)]
                    sa = plsc.bitcast(sw & MHI, jnp.float32)
                    da = plsc.bitcast(dw & MHI, jnp.float32)
                    sb = plsc.bitcast(lax.shift_left(sw, 16), jnp.float32)
                    db = plsc.bitcast(lax.shift_left(dw, 16), jnp.float32)
                    a = sa * da if a is None else a + sa * da
                    b = sb * db if b is None else b + sb * db
                return a + b

            def pair_body(p, _):
                e0 = 2 * p
                pbuf[p] = merge(edge_acc(e0), edge_acc(e0 + 1), 0)
                return 0

            def fin_body(g, _):
                q0 = g * 8
                a = merge(pbuf[q0 + 0], pbuf[q0 + 1], 1)
                b = merge(pbuf[q0 + 2], pbuf[q0 + 3], 1)
                c = merge(pbuf[q0 + 4], pbuf[q0 + 5], 1)
                d = merge(pbuf[q0 + 6], pbuf[q0 + 7], 1)
                ov[pl.ds(g * L, L)] = merge(merge(a, b, 2), merge(c, d, 2), 3)
                return 0

            lax.fori_loop(0, CH // 2, pair_body, 0)
            lax.fori_loop(0, CH // L, fin_body, 0)

        sid = lax.axis_index("s")
        rpt = NP // 16
        pltpu.sync_copy(x_hbm.at[pl.ds(sid * rpt, rpt)],
                        x_sh.at[pl.ds(sid * rpt, rpt)])
        plsc.subcore_barrier()

        issue_idx(0, 0)
        issue_idx(1, 1)
        wait_idx(0)
        issue_gather(0)

        def pair_body(i, carry):
            for b in (0, 1):
                c = 2 * i + b
                nb = 1 - b
                wait_gather(b)

                @pl.when(c + 2 < CHUNKS)
                def _():
                    issue_idx(c + 2, b)

                @pl.when(c + 1 < CHUNKS)
                def _():
                    wait_idx(nb)
                    issue_gather(nb)

                @pl.when(c >= 2)
                def _():
                    wait_out(b)

                compute(b)
                issue_out(c, b)
            return carry

        lax.fori_loop(0, CHUNKS // 2, pair_body, 0)
        wait_out(0)
        wait_out(1)

    return ip


_ip_kernel = _make_ip_kernel()


@jax.jit
def kernel(x, edge_index):
    ei = edge_index.astype(jnp.int32)
    src = jnp.pad(ei[0], (0, EP - E)).reshape(TCH, CH)
    dst = jnp.pad(ei[1], (0, EP - E)).reshape(TCH, CH)
    idx_packed = jnp.stack([src, dst], axis=1)
    x_bf = jnp.pad(x, ((0, NP - N), (0, 0))).astype(jnp.bfloat16)
    x_i32 = lax.bitcast_convert_type(x_bf.reshape(NP, D2, 2), jnp.int32)
    out = _ip_kernel(x_i32, idx_packed)
    return out[:E]
